# baseline (speedup 1.0000x reference)
import functools

import jax
import jax.numpy as jnp
from jax import lax
from jax.experimental import pallas as pl
from jax.experimental.pallas import tpu as pltpu
from jax.experimental.pallas import tpu_sc as plsc

N_NODES = 10000
N_PAD = 10240
N_EDGES = 320000
IN_DIM = 128
HID = 64
OUT = 2

NC = 2
NS = 16
NW = NC * NS
EDGES_PER_TILE = N_EDGES // NW
CHUNK_E = 400
N_STEPS_E = EDGES_PER_TILE // CHUNK_E
ROWS_PER_TILE = N_PAD // NS


def _zero_vmem(ref, n_rows, width):
  nv = width // 16

  def body(i, _):
    for j in range(nv):
      ref[i, pl.ds(j * 16, 16)] = jnp.zeros((16,), jnp.float32)
    return 0

  lax.fori_loop(0, n_rows, body, 0)


def _make_scatter_kernel(width):
  mesh = plsc.VectorSubcoreMesh(core_axis_name="c", subcore_axis_name="s")
  chunk = 400
  n_steps = EDGES_PER_TILE // chunk
  half = ROWS_PER_TILE // 2

  @functools.partial(
      pl.kernel,
      out_type=jax.ShapeDtypeStruct((NC, N_PAD, width), jnp.float32),
      mesh=mesh,
      compiler_params=pltpu.CompilerParams(use_tc_tiling_on_sc=False),
      scratch_types=[
          pltpu.VMEM_SHARED((N_PAD, width), jnp.float32),
          pltpu.VMEM((2, chunk), jnp.int32),
          pltpu.VMEM((2, chunk), jnp.int32),
          pltpu.VMEM((chunk, width), jnp.float32),
          pltpu.VMEM((chunk, width), jnp.float32),
          pltpu.SemaphoreType.DMA,
          pltpu.SemaphoreType.DMA,
          pltpu.SemaphoreType.DMA,
          pltpu.SemaphoreType.DMA,
      ],
  )
  def scatter_kernel(p_hbm, ei_hbm, out_hbm,
                     acc_sh, ib0, ib1, buf0, buf1, g0, g1, w0, w1):
    c = lax.axis_index("c")
    s = lax.axis_index("s")
    ib = (ib0, ib1)
    buf = (buf0, buf1)
    gsem = (g0, g1)
    wsem = (w0, w1)

    _zero_vmem(buf0, half, width)
    stage = buf0.at[pl.ds(0, half)]
    for h in range(2):
      pltpu.sync_copy(stage, acc_sh.at[pl.ds(s * ROWS_PER_TILE + h * half, half)])
    plsc.subcore_barrier()

    base = (c * NS + s) * EDGES_PER_TILE

    def load_idx(k, b):
      pltpu.sync_copy(ei_hbm.at[:, pl.ds(base + k * chunk, chunk)], ib[b])

    def start_gather(b):
      return pltpu.async_copy(p_hbm.at[ib[b].at[0]], buf[b], gsem[b])

    def start_scatter(b):
      return pltpu.async_copy(buf[b], acc_sh.at[ib[b].at[1]], wsem[b], add=True)

    load_idx(0, 0)
    gcp = [start_gather(0), None]
    scp = [None, None]
    for k in range(n_steps):
      b = k % 2
      nb = 1 - b
      if k + 1 < n_steps:
        if scp[nb] is not None:
          scp[nb].wait()
        load_idx(k + 1, nb)
        gcp[nb] = start_gather(nb)
      gcp[b].wait()
      scp[b] = start_scatter(b)
    scp[0].wait()
    scp[1].wait()
    plsc.subcore_barrier()

    for h in range(2):
      pltpu.sync_copy(acc_sh.at[pl.ds(s * ROWS_PER_TILE + h * half, half)], stage)
      pltpu.sync_copy(stage, out_hbm.at[c, pl.ds(s * ROWS_PER_TILE + h * half, half)])

  return scatter_kernel


def _make_edge_kernel():
  mesh = plsc.VectorSubcoreMesh(core_axis_name="c", subcore_axis_name="s")
  chunk = CHUNK_E
  n_steps = EDGES_PER_TILE // chunk
  NQ = HID // 16

  @functools.partial(
      pl.kernel,
      out_type=jax.ShapeDtypeStruct((OUT, N_EDGES), jnp.float32),
      mesh=mesh,
      compiler_params=pltpu.CompilerParams(use_tc_tiling_on_sc=False, needs_layout_passes=False),
      scratch_types=[
          pltpu.VMEM((2, chunk), jnp.int32),
          pltpu.VMEM((2, chunk), jnp.int32),
          pltpu.VMEM((chunk, HID), jnp.float32),
          pltpu.VMEM((chunk, HID), jnp.float32),
          pltpu.VMEM((chunk, HID), jnp.float32),
          pltpu.VMEM((chunk, HID), jnp.float32),
          pltpu.VMEM((OUT, chunk), jnp.float32),
          pltpu.VMEM((OUT, chunk), jnp.float32),
          pltpu.VMEM((OUT, HID), jnp.float32),
          pltpu.VMEM((16,), jnp.float32),
          pltpu.SemaphoreType.DMA,
          pltpu.SemaphoreType.DMA,
          pltpu.SemaphoreType.DMA,
          pltpu.SemaphoreType.DMA,
          pltpu.SemaphoreType.DMA,
          pltpu.SemaphoreType.DMA,
      ],
  )
  def edge_kernel(a_hbm, b_hbm, ei_hbm, w2_hbm, bm2_hbm, out_hbm,
                  ib0, ib1, ra0, ra1, rb0, rb1, ob0, ob1, wbuf, bbuf,
                  ga0, ga1, gb0, gb1, w0, w1):
    c = lax.axis_index("c")
    s = lax.axis_index("s")
    ib = (ib0, ib1)
    ra = (ra0, ra1)
    rb = (rb0, rb1)
    ob = (ob0, ob1)
    gsa = (ga0, ga1)
    gsb = (gb0, gb1)
    wsem = (w0, w1)
    base = (c * NS + s) * EDGES_PER_TILE

    pltpu.sync_copy(w2_hbm, wbuf)
    pltpu.sync_copy(bm2_hbm, bbuf)
    w = [[wbuf[j, pl.ds(q * 16, 16)] for q in range(NQ)] for j in range(OUT)]
    lane = lax.iota(jnp.int32, 16)
    bb = bbuf[...]
    bias = [jnp.full((16,), bb[j], jnp.float32) for j in range(OUT)]
    m15 = lane == 15

    def load_idx(k, b):
      pltpu.sync_copy(ei_hbm.at[:, pl.ds(base + k * chunk, chunk)], ib[b])

    def start_gathers(b):
      return (pltpu.async_copy(a_hbm.at[ib[b].at[0]], ra[b], gsa[b]),
              pltpu.async_copy(b_hbm.at[ib[b].at[1]], rb[b], gsb[b]))

    def compute(b):
      @plsc.parallel_loop(0, chunk, step=1, unroll=6)
      def body(r):
        h = [jnp.maximum(ra[b][r, pl.ds(q * 16, 16)] + rb[b][r, pl.ds(q * 16, 16)], 0.0)
             for q in range(NQ)]
        for j in range(OUT):
          t01 = h[0] * w[j][0] + h[1] * w[j][1]
          t23 = h[2] * w[j][2] + h[3] * w[j][3]
          cj = plsc.cumsum(t01 + t23) + bias[j]
          ridx = jnp.full((16,), j, jnp.int32)
          cidx = jnp.full((16,), r, jnp.int32)
          plsc.store_scatter(ob[b], [ridx, cidx], cj, mask=m15)

    load_idx(0, 0)
    gcp = [start_gathers(0), None]
    wcp = [None, None]
    for k in range(n_steps):
      b = k % 2
      nb = 1 - b
      if k + 1 < n_steps:
        if wcp[nb] is not None:
          wcp[nb].wait()
        load_idx(k + 1, nb)
        gcp[nb] = start_gathers(nb)
      gcp[b][0].wait()
      gcp[b][1].wait()
      compute(b)
      wcp[b] = pltpu.async_copy(
          ob[b], out_hbm.at[:, pl.ds(base + k * chunk, chunk)], wsem[b])
    wcp[0].wait()
    wcp[1].wait()

  return edge_kernel


_NODE_BLK = 1024
_NODE_GRID = N_PAD // _NODE_BLK
_EDGE_BLK = 4000
_EDGE_GRID = N_EDGES // _EDGE_BLK


def _full(shape):
  return pl.BlockSpec(shape, lambda i: tuple(0 for _ in shape))


def _tc1_body(x_ref, wl_ref, wr_ref, bl_ref, pext_ref, r_ref):
  x = x_ref[...]
  p = jnp.dot(x, wl_ref[...], preferred_element_type=jnp.float32)
  ones = jnp.ones((x.shape[0], 16), jnp.float32)
  pext_ref[...] = jnp.concatenate([p, ones], axis=1)
  r_ref[...] = jnp.dot(x, wr_ref[...], preferred_element_type=jnp.float32) + bl_ref[...]


def _tc2_body(agg_ref, r1_ref, wl_ref, wr_ref, bl_ref,
              p2_ref, r2_ref, inv_ref):
  a0 = agg_ref[0]
  a1 = agg_ref[1]
  cnt = a0[:, HID:HID + 1] + a1[:, HID:HID + 1]
  inv = 1.0 / jnp.maximum(cnt, 1.0)
  mean = (a0[:, :HID] + a1[:, :HID]) * inv
  h1 = jnp.maximum(mean + r1_ref[...], 0.0)
  p2_ref[...] = jnp.dot(h1, wl_ref[...], preferred_element_type=jnp.float32)
  r2_ref[...] = jnp.dot(h1, wr_ref[...], preferred_element_type=jnp.float32) + bl_ref[...]
  inv_ref[...] = inv


def _tc3_body(agg_ref, r2_ref, inv_ref, wa_ref, wb_ref, ba_ref,
              a_out_ref, b_out_ref):
  mean = (agg_ref[0] + agg_ref[1]) * inv_ref[...]
  h2 = jnp.maximum(mean + r2_ref[...], 0.0)
  a_out_ref[...] = jnp.dot(h2, wa_ref[...], preferred_element_type=jnp.float32) + ba_ref[...]
  b_out_ref[...] = jnp.dot(h2, wb_ref[...], preferred_element_type=jnp.float32)


def _tc4_body(g_ref, wm_ref, bm_ref, out_ref):
  h = jnp.maximum(g_ref[...], 0.0)
  out_ref[...] = jnp.dot(h, wm_ref[...], preferred_element_type=jnp.float32) + bm_ref[...]


def kernel(x, edge_index, Wl1, Wr1, bl1, Wl2, Wr2, bl2, Wm1, bm1, Wm2, bm2):
  ei = edge_index.astype(jnp.int32)

  p1ext, r1 = pl.pallas_call(
      _tc1_body,
      grid=(_NODE_GRID,),
      in_specs=[
          pl.BlockSpec((_NODE_BLK, IN_DIM), lambda i: (i, 0)),
          _full((IN_DIM, HID)),
          _full((IN_DIM, HID)),
          _full((1, HID)),
      ],
      out_specs=[
          pl.BlockSpec((_NODE_BLK, HID + 16), lambda i: (i, 0)),
          pl.BlockSpec((_NODE_BLK, HID), lambda i: (i, 0)),
      ],
      out_shape=[
          jax.ShapeDtypeStruct((N_PAD, HID + 16), jnp.float32),
          jax.ShapeDtypeStruct((N_PAD, HID), jnp.float32),
      ],
  )(x, Wl1, Wr1, bl1.reshape(1, HID))

  agg1 = _make_scatter_kernel(HID + 16)(p1ext, ei)

  p2, r2, inv = pl.pallas_call(
      _tc2_body,
      grid=(_NODE_GRID,),
      in_specs=[
          pl.BlockSpec((NC, _NODE_BLK, HID + 16), lambda i: (0, i, 0)),
          pl.BlockSpec((_NODE_BLK, HID), lambda i: (i, 0)),
          _full((HID, HID)),
          _full((HID, HID)),
          _full((1, HID)),
      ],
      out_specs=[
          pl.BlockSpec((_NODE_BLK, HID), lambda i: (i, 0)),
          pl.BlockSpec((_NODE_BLK, HID), lambda i: (i, 0)),
          pl.BlockSpec((_NODE_BLK, 1), lambda i: (i, 0)),
      ],
      out_shape=[
          jax.ShapeDtypeStruct((N_PAD, HID), jnp.float32),
          jax.ShapeDtypeStruct((N_PAD, HID), jnp.float32),
          jax.ShapeDtypeStruct((N_PAD, 1), jnp.float32),
      ],
  )(agg1, r1, Wl2, Wr2, bl2.reshape(1, HID))

  agg2 = _make_scatter_kernel(HID)(p2, ei)

  A, B = pl.pallas_call(
      _tc3_body,
      grid=(_NODE_GRID,),
      in_specs=[
          pl.BlockSpec((NC, _NODE_BLK, HID), lambda i: (0, i, 0)),
          pl.BlockSpec((_NODE_BLK, HID), lambda i: (i, 0)),
          pl.BlockSpec((_NODE_BLK, 1), lambda i: (i, 0)),
          _full((HID, HID)),
          _full((HID, HID)),
          _full((1, HID)),
      ],
      out_specs=[
          pl.BlockSpec((_NODE_BLK, HID), lambda i: (i, 0)),
          pl.BlockSpec((_NODE_BLK, HID), lambda i: (i, 0)),
      ],
      out_shape=[
          jax.ShapeDtypeStruct((N_PAD, HID), jnp.float32),
          jax.ShapeDtypeStruct((N_PAD, HID), jnp.float32),
      ],
  )(agg2, r2, inv, Wm1[:HID], Wm1[HID:], bm1.reshape(1, HID))

  out_t = _make_edge_kernel()(
      A, B, ei, jnp.transpose(Wm2), jnp.pad(bm2, (0, 16 - OUT)))
  return jnp.transpose(out_t)


# speedup vs baseline: 1.0126x; 1.0126x over previous
import functools

import jax
import jax.numpy as jnp
from jax import lax
from jax.experimental import pallas as pl
from jax.experimental.pallas import tpu as pltpu
from jax.experimental.pallas import tpu_sc as plsc

N_NODES = 10000
N_PAD = 10240
N_EDGES = 320000
IN_DIM = 128
HID = 64
OUT = 2

NC = 2
NS = 16
NW = NC * NS
EDGES_PER_TILE = N_EDGES // NW
CHUNK_E = 400
N_STEPS_E = EDGES_PER_TILE // CHUNK_E
ROWS_PER_TILE = N_PAD // NS


def _zero_vmem(ref, n_rows, width):
  nv = width // 16

  def body(i, _):
    for j in range(nv):
      ref[i, pl.ds(j * 16, 16)] = jnp.zeros((16,), jnp.float32)
    return 0

  lax.fori_loop(0, n_rows, body, 0)


def _make_scatter_kernel(width):
  mesh = plsc.VectorSubcoreMesh(core_axis_name="c", subcore_axis_name="s")
  chunk = 400
  n_steps = EDGES_PER_TILE // chunk
  half = ROWS_PER_TILE // 2

  @functools.partial(
      pl.kernel,
      out_type=jax.ShapeDtypeStruct((NC, N_PAD, width), jnp.float32),
      mesh=mesh,
      compiler_params=pltpu.CompilerParams(use_tc_tiling_on_sc=False),
      scratch_types=[
          pltpu.VMEM_SHARED((N_PAD, width), jnp.float32),
          pltpu.VMEM((2, chunk), jnp.int32),
          pltpu.VMEM((2, chunk), jnp.int32),
          pltpu.VMEM((chunk, width), jnp.float32),
          pltpu.VMEM((chunk, width), jnp.float32),
          pltpu.SemaphoreType.DMA,
          pltpu.SemaphoreType.DMA,
          pltpu.SemaphoreType.DMA,
          pltpu.SemaphoreType.DMA,
      ],
  )
  def scatter_kernel(p_hbm, ei_hbm, out_hbm,
                     acc_sh, ib0, ib1, buf0, buf1, g0, g1, w0, w1):
    c = lax.axis_index("c")
    s = lax.axis_index("s")
    ib = (ib0, ib1)
    buf = (buf0, buf1)
    gsem = (g0, g1)
    wsem = (w0, w1)

    _zero_vmem(buf0, half, width)
    stage = buf0.at[pl.ds(0, half)]
    for h in range(2):
      pltpu.sync_copy(stage, acc_sh.at[pl.ds(s * ROWS_PER_TILE + h * half, half)])
    plsc.subcore_barrier()

    base = (c * NS + s) * EDGES_PER_TILE

    def load_idx(k, b):
      pltpu.sync_copy(ei_hbm.at[:, pl.ds(base + k * chunk, chunk)], ib[b])

    def start_gather(b):
      return pltpu.async_copy(p_hbm.at[ib[b].at[0]], buf[b], gsem[b])

    def start_scatter(b):
      return pltpu.async_copy(buf[b], acc_sh.at[ib[b].at[1]], wsem[b], add=True)

    load_idx(0, 0)
    gcp = [start_gather(0), None]
    scp = [None, None]
    for k in range(n_steps):
      b = k % 2
      nb = 1 - b
      if k + 1 < n_steps:
        if scp[nb] is not None:
          scp[nb].wait()
        load_idx(k + 1, nb)
        gcp[nb] = start_gather(nb)
      gcp[b].wait()
      scp[b] = start_scatter(b)
    scp[0].wait()
    scp[1].wait()
    plsc.subcore_barrier()

    for h in range(2):
      pltpu.sync_copy(acc_sh.at[pl.ds(s * ROWS_PER_TILE + h * half, half)], stage)
      pltpu.sync_copy(stage, out_hbm.at[c, pl.ds(s * ROWS_PER_TILE + h * half, half)])

  return scatter_kernel


def _make_edge_kernel():
  mesh = plsc.VectorSubcoreMesh(core_axis_name="c", subcore_axis_name="s")
  chunk = CHUNK_E
  n_steps = EDGES_PER_TILE // chunk
  NQ = HID // 16

  @functools.partial(
      pl.kernel,
      out_type=jax.ShapeDtypeStruct((OUT, N_EDGES), jnp.float32),
      mesh=mesh,
      compiler_params=pltpu.CompilerParams(use_tc_tiling_on_sc=False, needs_layout_passes=False),
      scratch_types=[
          pltpu.VMEM((2, chunk), jnp.int32),
          pltpu.VMEM((2, chunk), jnp.int32),
          pltpu.VMEM((chunk, HID), jnp.float32),
          pltpu.VMEM((chunk, HID), jnp.float32),
          pltpu.VMEM((chunk, HID), jnp.float32),
          pltpu.VMEM((chunk, HID), jnp.float32),
          pltpu.VMEM((OUT, chunk), jnp.float32),
          pltpu.VMEM((OUT, chunk), jnp.float32),
          pltpu.VMEM((OUT, HID), jnp.float32),
          pltpu.VMEM((16,), jnp.float32),
          pltpu.SemaphoreType.DMA,
          pltpu.SemaphoreType.DMA,
          pltpu.SemaphoreType.DMA,
          pltpu.SemaphoreType.DMA,
          pltpu.SemaphoreType.DMA,
          pltpu.SemaphoreType.DMA,
      ],
  )
  def edge_kernel(a_hbm, b_hbm, ei_hbm, w2_hbm, bm2_hbm, out_hbm,
                  ib0, ib1, ra0, ra1, rb0, rb1, ob0, ob1, wbuf, bbuf,
                  ga0, ga1, gb0, gb1, w0, w1):
    c = lax.axis_index("c")
    s = lax.axis_index("s")
    ib = (ib0, ib1)
    ra = (ra0, ra1)
    rb = (rb0, rb1)
    ob = (ob0, ob1)
    gsa = (ga0, ga1)
    gsb = (gb0, gb1)
    wsem = (w0, w1)
    base = (c * NS + s) * EDGES_PER_TILE

    pltpu.sync_copy(w2_hbm, wbuf)
    pltpu.sync_copy(bm2_hbm, bbuf)
    w = [[wbuf[j, pl.ds(q * 16, 16)] for q in range(NQ)] for j in range(OUT)]
    lane = lax.iota(jnp.int32, 16)
    bb = bbuf[...]
    bias = [jnp.full((16,), bb[j], jnp.float32) for j in range(OUT)]
    m15 = lane == 15

    def load_idx(k, b):
      pltpu.sync_copy(ei_hbm.at[:, pl.ds(base + k * chunk, chunk)], ib[b])

    def start_gathers(b):
      return (pltpu.async_copy(a_hbm.at[ib[b].at[0]], ra[b], gsa[b]),
              pltpu.async_copy(b_hbm.at[ib[b].at[1]], rb[b], gsb[b]))

    def compute(b):
      @plsc.parallel_loop(0, chunk, step=1, unroll=4)
      def body(r):
        h = [jnp.maximum(ra[b][r, pl.ds(q * 16, 16)] + rb[b][r, pl.ds(q * 16, 16)], 0.0)
             for q in range(NQ)]
        for j in range(OUT):
          t01 = h[0] * w[j][0] + h[1] * w[j][1]
          t23 = h[2] * w[j][2] + h[3] * w[j][3]
          cj = plsc.cumsum(t01 + t23) + bias[j]
          ridx = jnp.full((16,), j, jnp.int32)
          cidx = jnp.full((16,), r, jnp.int32)
          plsc.store_scatter(ob[b], [ridx, cidx], cj, mask=m15)

    load_idx(0, 0)
    gcp = [start_gathers(0), None]
    wcp = [None, None]
    for k in range(n_steps):
      b = k % 2
      nb = 1 - b
      if k + 1 < n_steps:
        if wcp[nb] is not None:
          wcp[nb].wait()
        load_idx(k + 1, nb)
        gcp[nb] = start_gathers(nb)
      gcp[b][0].wait()
      gcp[b][1].wait()
      compute(b)
      wcp[b] = pltpu.async_copy(
          ob[b], out_hbm.at[:, pl.ds(base + k * chunk, chunk)], wsem[b])
    wcp[0].wait()
    wcp[1].wait()

  return edge_kernel


_NODE_BLK = 1024
_NODE_GRID = N_PAD // _NODE_BLK
_EDGE_BLK = 4000
_EDGE_GRID = N_EDGES // _EDGE_BLK


def _full(shape):
  return pl.BlockSpec(shape, lambda i: tuple(0 for _ in shape))


def _tc1_body(x_ref, wl_ref, wr_ref, bl_ref, pext_ref, r_ref):
  x = x_ref[...]
  p = jnp.dot(x, wl_ref[...], preferred_element_type=jnp.float32)
  ones = jnp.ones((x.shape[0], 16), jnp.float32)
  pext_ref[...] = jnp.concatenate([p, ones], axis=1)
  r_ref[...] = jnp.dot(x, wr_ref[...], preferred_element_type=jnp.float32) + bl_ref[...]


def _tc2_body(agg_ref, r1_ref, wl_ref, wr_ref, bl_ref,
              p2_ref, r2_ref, inv_ref):
  a0 = agg_ref[0]
  a1 = agg_ref[1]
  cnt = a0[:, HID:HID + 1] + a1[:, HID:HID + 1]
  inv = 1.0 / jnp.maximum(cnt, 1.0)
  mean = (a0[:, :HID] + a1[:, :HID]) * inv
  h1 = jnp.maximum(mean + r1_ref[...], 0.0)
  p2_ref[...] = jnp.dot(h1, wl_ref[...], preferred_element_type=jnp.float32)
  r2_ref[...] = jnp.dot(h1, wr_ref[...], preferred_element_type=jnp.float32) + bl_ref[...]
  inv_ref[...] = inv


def _tc3_body(agg_ref, r2_ref, inv_ref, wa_ref, wb_ref, ba_ref,
              a_out_ref, b_out_ref):
  mean = (agg_ref[0] + agg_ref[1]) * inv_ref[...]
  h2 = jnp.maximum(mean + r2_ref[...], 0.0)
  a_out_ref[...] = jnp.dot(h2, wa_ref[...], preferred_element_type=jnp.float32) + ba_ref[...]
  b_out_ref[...] = jnp.dot(h2, wb_ref[...], preferred_element_type=jnp.float32)


def _tc4_body(g_ref, wm_ref, bm_ref, out_ref):
  h = jnp.maximum(g_ref[...], 0.0)
  out_ref[...] = jnp.dot(h, wm_ref[...], preferred_element_type=jnp.float32) + bm_ref[...]


def kernel(x, edge_index, Wl1, Wr1, bl1, Wl2, Wr2, bl2, Wm1, bm1, Wm2, bm2):
  ei = edge_index.astype(jnp.int32)

  p1ext, r1 = pl.pallas_call(
      _tc1_body,
      grid=(_NODE_GRID,),
      in_specs=[
          pl.BlockSpec((_NODE_BLK, IN_DIM), lambda i: (i, 0)),
          _full((IN_DIM, HID)),
          _full((IN_DIM, HID)),
          _full((1, HID)),
      ],
      out_specs=[
          pl.BlockSpec((_NODE_BLK, HID + 16), lambda i: (i, 0)),
          pl.BlockSpec((_NODE_BLK, HID), lambda i: (i, 0)),
      ],
      out_shape=[
          jax.ShapeDtypeStruct((N_PAD, HID + 16), jnp.float32),
          jax.ShapeDtypeStruct((N_PAD, HID), jnp.float32),
      ],
  )(x, Wl1, Wr1, bl1.reshape(1, HID))

  agg1 = _make_scatter_kernel(HID + 16)(p1ext, ei)

  p2, r2, inv = pl.pallas_call(
      _tc2_body,
      grid=(_NODE_GRID,),
      in_specs=[
          pl.BlockSpec((NC, _NODE_BLK, HID + 16), lambda i: (0, i, 0)),
          pl.BlockSpec((_NODE_BLK, HID), lambda i: (i, 0)),
          _full((HID, HID)),
          _full((HID, HID)),
          _full((1, HID)),
      ],
      out_specs=[
          pl.BlockSpec((_NODE_BLK, HID), lambda i: (i, 0)),
          pl.BlockSpec((_NODE_BLK, HID), lambda i: (i, 0)),
          pl.BlockSpec((_NODE_BLK, 1), lambda i: (i, 0)),
      ],
      out_shape=[
          jax.ShapeDtypeStruct((N_PAD, HID), jnp.float32),
          jax.ShapeDtypeStruct((N_PAD, HID), jnp.float32),
          jax.ShapeDtypeStruct((N_PAD, 1), jnp.float32),
      ],
  )(agg1, r1, Wl2, Wr2, bl2.reshape(1, HID))

  agg2 = _make_scatter_kernel(HID)(p2, ei)

  A, B = pl.pallas_call(
      _tc3_body,
      grid=(_NODE_GRID,),
      in_specs=[
          pl.BlockSpec((NC, _NODE_BLK, HID), lambda i: (0, i, 0)),
          pl.BlockSpec((_NODE_BLK, HID), lambda i: (i, 0)),
          pl.BlockSpec((_NODE_BLK, 1), lambda i: (i, 0)),
          _full((HID, HID)),
          _full((HID, HID)),
          _full((1, HID)),
      ],
      out_specs=[
          pl.BlockSpec((_NODE_BLK, HID), lambda i: (i, 0)),
          pl.BlockSpec((_NODE_BLK, HID), lambda i: (i, 0)),
      ],
      out_shape=[
          jax.ShapeDtypeStruct((N_PAD, HID), jnp.float32),
          jax.ShapeDtypeStruct((N_PAD, HID), jnp.float32),
      ],
  )(agg2, r2, inv, Wm1[:HID], Wm1[HID:], bm1.reshape(1, HID))

  out_t = _make_edge_kernel()(
      A, B, ei, jnp.transpose(Wm2), jnp.pad(bm2, (0, 16 - OUT)))
  return jnp.transpose(out_t)
